# baseline (speedup 1.0000x reference)
import jax
import jax.numpy as jnp
from jax import lax
from jax.experimental import pallas as pl
from jax.experimental.pallas import tpu as pltpu
from jax.experimental.pallas import tpu_sc as plsc

D = 32
SEQ = 50
B = 16384
L = 16
NC, NS = 2, 16
NW = NC * NS
SENT_PER_W = B // NW
CHS = 32
NCHUNK = SENT_PER_W // CHS
TOK = CHS * SEQ


def _body(ids_hbm, table_hbm, out_hbm, idx0, idx1, rows0, rows1, out_v,
          sem0, sem1):
    wid = lax.axis_index("s") * NC + lax.axis_index("c")
    tok_base = wid * (SENT_PER_W * SEQ)
    sent_base = wid * SENT_PER_W

    def fire(c, idx_v, rows_v, sem):
        pltpu.sync_copy(ids_hbm.at[pl.ds(tok_base + c * TOK, TOK)], idx_v)
        pltpu.async_copy(table_hbm.at[idx_v], rows_v, sem)

    def drain_and_pool(c, idx_v, rows_v, sem):
        pltpu.make_async_copy(table_hbm.at[idx_v], rows_v, sem).wait()

        def sent(s, carry):
            base = s * SEQ
            acc0 = rows_v[base, pl.ds(0, L)]
            acc1 = rows_v[base, pl.ds(L, L)]
            for r in range(1, SEQ):
                acc0 = acc0 + rows_v[base + r, pl.ds(0, L)]
                acc1 = acc1 + rows_v[base + r, pl.ds(L, L)]
            out_v[s, pl.ds(0, L)] = acc0 * (1.0 / SEQ)
            out_v[s, pl.ds(L, L)] = acc1 * (1.0 / SEQ)
            return carry

        lax.fori_loop(0, CHS, sent, 0)
        pltpu.sync_copy(out_v, out_hbm.at[pl.ds(sent_base + c * CHS, CHS)])

    fire(0, idx0, rows0, sem0)

    def pair(i, carry):
        a = 2 * i
        b = a + 1
        fire(b, idx1, rows1, sem1)
        drain_and_pool(a, idx0, rows0, sem0)

        @pl.when(b + 1 < NCHUNK)
        def _():
            fire(b + 1, idx0, rows0, sem0)

        drain_and_pool(b, idx1, rows1, sem1)
        return carry

    lax.fori_loop(0, NCHUNK // 2, pair, 0)


def _tc_busy_body(x_ref, o_ref):
    def it(i, v):
        return v * 1.0000001 + 1e-9
    o_ref[...] = lax.fori_loop(0, 200000, it, x_ref[...])


def kernel(token_ids, table):
    ids = token_ids.astype(jnp.int32).reshape(B * SEQ)
    tc_busy = pl.pallas_call(
        _tc_busy_body,
        out_shape=jax.ShapeDtypeStruct((8, 128), jnp.float32),
    )(table[:32].reshape(8, 128))
    mesh = plsc.VectorSubcoreMesh(
        core_axis_name="c", subcore_axis_name="s", num_cores=NC, num_subcores=NS
    )
    f = pl.kernel(
        _body,
        out_type=jax.ShapeDtypeStruct((B, D), jnp.float32),
        mesh=mesh,
        scratch_types=[
            pltpu.VMEM((TOK,), jnp.int32),
            pltpu.VMEM((TOK,), jnp.int32),
            pltpu.VMEM((TOK, D), jnp.float32),
            pltpu.VMEM((TOK, D), jnp.float32),
            pltpu.VMEM((CHS, D), jnp.float32),
            pltpu.SemaphoreType.DMA,
            pltpu.SemaphoreType.DMA,
        ],
        compiler_params=pltpu.CompilerParams(use_tc_tiling_on_sc=False),
    )
    out = f(ids, table)
    return out + 0.0 * tc_busy[0, 0]


# speedup vs baseline: 2.9347x; 2.9347x over previous
import jax
import jax.numpy as jnp
from jax import lax
from jax.experimental import pallas as pl
from jax.experimental.pallas import tpu as pltpu
from jax.experimental.pallas import tpu_sc as plsc

D = 32
SEQ = 50
B = 16384
L = 16
NC, NS = 2, 16
NW = NC * NS
SENT_PER_W = B // NW
CHS = 32
NCHUNK = SENT_PER_W // CHS
TOK = CHS * SEQ


def _body(ids_hbm, table_hbm, out_hbm, idx0, idx1, rows0, rows1, out_v,
          sem0, sem1):
    wid = lax.axis_index("s") * NC + lax.axis_index("c")
    tok_base = wid * (SENT_PER_W * SEQ)
    sent_base = wid * SENT_PER_W

    def fire(c, idx_v, rows_v, sem):
        pltpu.sync_copy(ids_hbm.at[pl.ds(tok_base + c * TOK, TOK)], idx_v)
        pltpu.async_copy(table_hbm.at[idx_v], rows_v, sem)

    def drain_and_pool(c, idx_v, rows_v, sem):
        pltpu.make_async_copy(table_hbm.at[idx_v], rows_v, sem).wait()

        def sent(s, carry):
            base = s * SEQ
            acc0 = rows_v[base, pl.ds(0, L)]
            acc1 = rows_v[base, pl.ds(L, L)]
            for r in range(1, SEQ):
                acc0 = acc0 + rows_v[base + r, pl.ds(0, L)]
                acc1 = acc1 + rows_v[base + r, pl.ds(L, L)]
            out_v[s, pl.ds(0, L)] = acc0 * (1.0 / SEQ)
            out_v[s, pl.ds(L, L)] = acc1 * (1.0 / SEQ)
            return carry

        lax.fori_loop(0, CHS, sent, 0)
        pltpu.sync_copy(out_v, out_hbm.at[pl.ds(sent_base + c * CHS, CHS)])

    fire(0, idx0, rows0, sem0)

    def pair(i, carry):
        a = 2 * i
        b = a + 1
        fire(b, idx1, rows1, sem1)
        drain_and_pool(a, idx0, rows0, sem0)

        @pl.when(b + 1 < NCHUNK)
        def _():
            fire(b + 1, idx0, rows0, sem0)

        drain_and_pool(b, idx1, rows1, sem1)
        return carry

    lax.fori_loop(0, NCHUNK // 2, pair, 0)


def kernel(token_ids, table):
    ids = token_ids.astype(jnp.int32).reshape(B * SEQ)
    mesh = plsc.VectorSubcoreMesh(
        core_axis_name="c", subcore_axis_name="s", num_cores=NC, num_subcores=NS
    )
    f = pl.kernel(
        _body,
        out_type=jax.ShapeDtypeStruct((B, D), jnp.float32),
        mesh=mesh,
        scratch_types=[
            pltpu.VMEM((TOK,), jnp.int32),
            pltpu.VMEM((TOK,), jnp.int32),
            pltpu.VMEM((TOK, D), jnp.float32),
            pltpu.VMEM((TOK, D), jnp.float32),
            pltpu.VMEM((CHS, D), jnp.float32),
            pltpu.SemaphoreType.DMA,
            pltpu.SemaphoreType.DMA,
        ],
        compiler_params=pltpu.CompilerParams(use_tc_tiling_on_sc=False),
    )
    return f(ids, table)


# speedup vs baseline: 2.9449x; 1.0035x over previous
import jax
import jax.numpy as jnp
from jax import lax
from jax.experimental import pallas as pl
from jax.experimental.pallas import tpu as pltpu
from jax.experimental.pallas import tpu_sc as plsc

D = 32
SEQ = 50
B = 16384
L = 16
NC, NS = 2, 16
NW = NC * NS
SENT_PER_W = B // NW
CHS = 32
NCHUNK = SENT_PER_W // CHS
TOK = CHS * SEQ


def _body(ids_hbm, table_hbm, out_hbm, idx0, idx1, rows0, rows1, out_v,
          sem0, sem1):
    wid = lax.axis_index("s") * NC + lax.axis_index("c")
    tok_base = wid * (SENT_PER_W * SEQ)
    sent_base = wid * SENT_PER_W

    def fire(c, idx_v, rows_v, sem):
        pltpu.sync_copy(ids_hbm.at[pl.ds(tok_base + c * TOK, TOK)], idx_v)
        pltpu.async_copy(table_hbm.at[idx_v], rows_v, sem)

    def drain_and_pool(c, idx_v, rows_v, sem):
        pltpu.make_async_copy(table_hbm.at[idx_v], rows_v, sem).wait()

        def sent(s, carry):
            base = s * SEQ
            acc0 = rows_v[base, pl.ds(0, L)]
            acc1 = rows_v[base, pl.ds(L, L)]
            for r in range(1, SEQ):
                acc0 = acc0 + rows_v[base + r, pl.ds(0, L)]
                acc1 = acc1 + rows_v[base + r, pl.ds(L, L)]
            out_v[s, pl.ds(0, L)] = acc0 * (1.0 / SEQ)
            out_v[s, pl.ds(L, L)] = acc1 * (1.0 / SEQ)
            return carry

        lax.fori_loop(0, CHS, sent, 0)
        pltpu.sync_copy(out_v, out_hbm.at[pl.ds(sent_base + c * CHS, CHS)])

    fire(0, idx0, rows0, sem0)

    def pair(i, carry):
        a = 2 * i
        b = a + 1
        fire(b, idx1, rows1, sem1)
        drain_and_pool(a, idx0, rows0, sem0)

        @pl.when(b + 1 < NCHUNK)
        def _():
            fire(b + 1, idx0, rows0, sem0)

        drain_and_pool(b, idx1, rows1, sem1)
        return carry

    lax.fori_loop(0, NCHUNK // 2, pair, 0)


def kernel(token_ids, table):
    one = token_ids[0, 0] * 0 + 1
    ids = token_ids.astype(jnp.int32).reshape(B * SEQ) * one
    mesh = plsc.VectorSubcoreMesh(
        core_axis_name="c", subcore_axis_name="s", num_cores=NC, num_subcores=NS
    )
    f = pl.kernel(
        _body,
        out_type=jax.ShapeDtypeStruct((B, D), jnp.float32),
        mesh=mesh,
        scratch_types=[
            pltpu.VMEM((TOK,), jnp.int32),
            pltpu.VMEM((TOK,), jnp.int32),
            pltpu.VMEM((TOK, D), jnp.float32),
            pltpu.VMEM((TOK, D), jnp.float32),
            pltpu.VMEM((CHS, D), jnp.float32),
            pltpu.SemaphoreType.DMA,
            pltpu.SemaphoreType.DMA,
        ],
        compiler_params=pltpu.CompilerParams(use_tc_tiling_on_sc=False),
    )
    return f(ids, table) * one.astype(jnp.float32)


# speedup vs baseline: 4.7974x; 1.6290x over previous
import jax
import jax.numpy as jnp
from jax import lax
from jax.experimental import pallas as pl
from jax.experimental.pallas import tpu as pltpu
from jax.experimental.pallas import tpu_sc as plsc

D = 32
SEQ = 50
B = 16384
L = 16
NC, NS = 2, 16
NW = NC * NS
SENT_PER_W = B // NW
CHS = 32
NCHUNK = SENT_PER_W // CHS
TOK = CHS * SEQ

TB = 8192
QS = TB // 4


def _tr_body(x_ref, o_ref):
    xt = jnp.transpose(x_ref[...])
    o_ref[...] = jnp.concatenate(
        [xt[q * QS:(q + 1) * QS, :] for q in range(4)], axis=1
    )


def _tc_linearize(table):
    v = table.shape[0]
    nb = pl.cdiv(v, TB)
    tt = jnp.swapaxes(table, 0, 1)
    lin = pl.pallas_call(
        _tr_body,
        grid=(nb,),
        in_specs=[pl.BlockSpec((D, TB), lambda i: (0, i))],
        out_specs=pl.BlockSpec((QS, 128), lambda i: (i, 0)),
        out_shape=jax.ShapeDtypeStruct((nb * QS, 128), jnp.float32),
    )(tt)
    return lin.reshape(nb * TB, D)


def _body(ids_hbm, table_hbm, out_hbm, idx0, idx1, rows0, rows1, out_v,
          sem0, sem1):
    wid = lax.axis_index("s") * NC + lax.axis_index("c")
    tok_base = wid * (SENT_PER_W * SEQ)
    sent_base = wid * SENT_PER_W

    def fire(c, idx_v, rows_v, sem):
        pltpu.sync_copy(ids_hbm.at[pl.ds(tok_base + c * TOK, TOK)], idx_v)

        def remap(k, carry):
            t = idx_v[pl.ds(k * L, L)]
            b = lax.bitwise_and(t, TB - 1)
            p = (lax.bitwise_and(t, ~(TB - 1))
                 + lax.shift_left(lax.bitwise_and(t, QS - 1), 2)
                 + lax.shift_right_logical(b, 11))
            idx_v[pl.ds(k * L, L)] = p
            return carry

        lax.fori_loop(0, TOK // L, remap, 0)
        pltpu.async_copy(table_hbm.at[idx_v], rows_v, sem)

    def drain_and_pool(c, idx_v, rows_v, sem):
        pltpu.make_async_copy(table_hbm.at[idx_v], rows_v, sem).wait()

        def sent(s, carry):
            base = s * SEQ
            acc0 = rows_v[base, pl.ds(0, L)]
            acc1 = rows_v[base, pl.ds(L, L)]
            for r in range(1, SEQ):
                acc0 = acc0 + rows_v[base + r, pl.ds(0, L)]
                acc1 = acc1 + rows_v[base + r, pl.ds(L, L)]
            out_v[s, pl.ds(0, L)] = acc0 * (1.0 / SEQ)
            out_v[s, pl.ds(L, L)] = acc1 * (1.0 / SEQ)
            return carry

        lax.fori_loop(0, CHS, sent, 0)
        pltpu.sync_copy(out_v, out_hbm.at[pl.ds(sent_base + c * CHS, CHS)])

    fire(0, idx0, rows0, sem0)

    def pair(i, carry):
        a = 2 * i
        b = a + 1
        fire(b, idx1, rows1, sem1)
        drain_and_pool(a, idx0, rows0, sem0)

        @pl.when(b + 1 < NCHUNK)
        def _():
            fire(b + 1, idx0, rows0, sem0)

        drain_and_pool(b, idx1, rows1, sem1)
        return carry

    lax.fori_loop(0, NCHUNK // 2, pair, 0)


def kernel(token_ids, table):
    ids = token_ids.astype(jnp.int32).reshape(B * SEQ)
    lin = _tc_linearize(table)
    mesh = plsc.VectorSubcoreMesh(
        core_axis_name="c", subcore_axis_name="s", num_cores=NC, num_subcores=NS
    )
    f = pl.kernel(
        _body,
        out_type=jax.ShapeDtypeStruct((B, D), jnp.float32),
        mesh=mesh,
        scratch_types=[
            pltpu.VMEM((TOK,), jnp.int32),
            pltpu.VMEM((TOK,), jnp.int32),
            pltpu.VMEM((TOK, D), jnp.float32),
            pltpu.VMEM((TOK, D), jnp.float32),
            pltpu.VMEM((CHS, D), jnp.float32),
            pltpu.SemaphoreType.DMA,
            pltpu.SemaphoreType.DMA,
        ],
        compiler_params=pltpu.CompilerParams(use_tc_tiling_on_sc=False),
    )
    return f(ids, lin)
